# Initial kernel scaffold; baseline (speedup 1.0000x reference)
import jax
import jax.numpy as jnp
from jax.experimental import pallas as pl


def kernel(x_text, x_image, x_other, text__w_lin, text__b_lin, text__w_ih_f, text__w_hh_f, text__b_ih_f, text__b_hh_f, text__w_ih_b, text__w_hh_b, text__b_ih_b, text__b_hh_b, image__w_lin, image__b_lin, image__w_ih_f, image__w_hh_f, image__b_ih_f, image__b_hh_f, image__w_ih_b, image__w_hh_b, image__b_ih_b, image__b_hh_b, other__w_lin, other__b_lin, other__w_ih_f, other__w_hh_f, other__b_ih_f, other__b_hh_f, other__w_ih_b, other__w_hh_b, other__b_ih_b, other__b_hh_b):
    raise NotImplementedError("write your pallas kernel here")


# speedup vs baseline: 1.2105x; 1.2105x over previous
import jax
import jax.numpy as jnp
from jax.experimental import pallas as pl
from jax.experimental.pallas import tpu as pltpu


def _bi_rnn_kernel(
        xt_ref, xi_ref, xo_ref,
        wlt_t_ref, wlt_i_ref, wlt_o_ref,
        bl_t_ref, bl_i_ref, bl_o_ref,
        wct_t_ref, wct_i_ref, wct_o_ref,
        bc_t_ref, bc_i_ref, bc_o_ref,
        whh_ref,
        out_ref,
        gt_scr, gi_scr, go_scr):
    Bh, T, two_h = out_ref.shape
    H = two_h // 2
    G = 4 * H

    def project(x_ref, wlt_ref, bl_ref, wct_ref, bc_ref, g_scr):
        TT, BB, D = x_ref.shape
        x2 = x_ref[...].reshape(TT * BB, D)
        lin = jnp.dot(x2, wlt_ref[...],
                      preferred_element_type=jnp.float32) + bl_ref[...]
        g_scr[...] = jnp.dot(lin.astype(jnp.bfloat16), wct_ref[...],
                             preferred_element_type=jnp.float32) + bc_ref[...]

    project(xt_ref, wlt_t_ref, bl_t_ref, wct_t_ref, bc_t_ref, gt_scr)
    project(xi_ref, wlt_i_ref, bl_i_ref, wct_i_ref, bc_i_ref, gi_scr)
    project(xo_ref, wlt_o_ref, bl_o_ref, wct_o_ref, bc_o_ref, go_scr)

    g_scrs = (gt_scr, gi_scr, go_scr)

    h = [jnp.zeros((Bh, H), jnp.float32) for _ in range(6)]
    c = [jnp.zeros((Bh, H), jnp.float32) for _ in range(6)]
    inv3 = 1.0 / 3.0

    for t in range(T):
        fwd_sum = jnp.zeros((Bh, H), jnp.float32)
        bwd_sum = jnp.zeros((Bh, H), jnp.float32)
        for m in range(3):
            for d in range(2):
                s = 2 * m + d
                tt = t if d == 0 else T - 1 - t
                gx = g_scrs[m][tt * Bh:(tt + 1) * Bh, d * G:(d + 1) * G]
                gsum = gx + jnp.dot(h[s].astype(jnp.bfloat16), whh_ref[s],
                                    preferred_element_type=jnp.float32)
                i_g = jax.nn.sigmoid(gsum[:, 0:H])
                f_g = jax.nn.sigmoid(gsum[:, H:2 * H])
                g_g = jnp.tanh(gsum[:, 2 * H:3 * H])
                o_g = jax.nn.sigmoid(gsum[:, 3 * H:4 * H])
                c[s] = f_g * c[s] + i_g * g_g
                h[s] = o_g * jnp.tanh(c[s])
                if d == 0:
                    fwd_sum = fwd_sum + h[s]
                else:
                    bwd_sum = bwd_sum + h[s]
        out_ref[:, t:t + 1, 0:H] = (fwd_sum * inv3)[:, None, :]
        out_ref[:, T - 1 - t:T - t, H:2 * H] = (bwd_sum * inv3)[:, None, :]


def _prep_modality(x, w_lin, b_lin, w_ih_f, w_ih_b, b_ih_f, b_hh_f,
                   b_ih_b, b_hh_b):
    B, T, D = x.shape
    x_tm = jnp.transpose(x, (1, 0, 2)).astype(jnp.bfloat16)
    wlt = jnp.transpose(w_lin).astype(jnp.bfloat16)
    bl = b_lin.reshape(1, D)
    wct = jnp.concatenate([jnp.transpose(w_ih_f),
                           jnp.transpose(w_ih_b)],
                          axis=1).astype(jnp.bfloat16)
    bc = jnp.concatenate([b_ih_f + b_hh_f,
                          b_ih_b + b_hh_b]).reshape(1, -1)
    return x_tm, wlt, bl, wct, bc


_NCORES = 2


@jax.jit
def _bi_rnn_forward(xs, mods):
    B, T, _ = xs[0].shape
    H = mods[0][3].shape[1]
    Bh = B // _NCORES

    packed = [_prep_modality(x, p[0], p[1], p[2], p[6], p[4], p[5], p[8], p[9])
              for x, p in zip(xs, mods)]
    xtm, wlts, bls, wcts, bcs = zip(*packed)

    whh_t = jnp.stack([jnp.transpose(p[k]) for p in mods for k in (3, 7)],
                      axis=0).astype(jnp.bfloat16)

    args = (*xtm, *wlts, *bls, *wcts, *bcs, whh_t)

    def full(shape):
        zeros = (0,) * len(shape)
        return pl.BlockSpec(shape, lambda i, _z=zeros: _z)

    in_specs = []
    for a in args[:3]:
        in_specs.append(pl.BlockSpec((T, Bh, a.shape[2]),
                                     lambda i: (0, i, 0)))
    for a in args[3:]:
        in_specs.append(full(a.shape))

    return pl.pallas_call(
        _bi_rnn_kernel,
        out_shape=jax.ShapeDtypeStruct((B, T, 2 * H), jnp.float32),
        grid=(_NCORES,),
        in_specs=in_specs,
        out_specs=pl.BlockSpec((Bh, T, 2 * H), lambda i: (i, 0, 0)),
        scratch_shapes=[pltpu.VMEM((T * Bh, 8 * H), jnp.float32)
                        for _ in range(3)],
        compiler_params=pltpu.CompilerParams(
            dimension_semantics=("parallel",)),
    )(*args)


def kernel(x_text, x_image, x_other,
           text__w_lin, text__b_lin,
           text__w_ih_f, text__w_hh_f, text__b_ih_f, text__b_hh_f,
           text__w_ih_b, text__w_hh_b, text__b_ih_b, text__b_hh_b,
           image__w_lin, image__b_lin,
           image__w_ih_f, image__w_hh_f, image__b_ih_f, image__b_hh_f,
           image__w_ih_b, image__w_hh_b, image__b_ih_b, image__b_hh_b,
           other__w_lin, other__b_lin,
           other__w_ih_f, other__w_hh_f, other__b_ih_f, other__b_hh_f,
           other__w_ih_b, other__w_hh_b, other__b_ih_b, other__b_hh_b):
    xs = (x_text, x_image, x_other)
    mods = (
        (text__w_lin, text__b_lin, text__w_ih_f, text__w_hh_f,
         text__b_ih_f, text__b_hh_f, text__w_ih_b, text__w_hh_b,
         text__b_ih_b, text__b_hh_b),
        (image__w_lin, image__b_lin, image__w_ih_f, image__w_hh_f,
         image__b_ih_f, image__b_hh_f, image__w_ih_b, image__w_hh_b,
         image__b_ih_b, image__b_hh_b),
        (other__w_lin, other__b_lin, other__w_ih_f, other__w_hh_f,
         other__b_ih_f, other__b_hh_f, other__w_ih_b, other__w_hh_b,
         other__b_ih_b, other__b_hh_b),
    )
    return _bi_rnn_forward(xs, mods)


# speedup vs baseline: 2.0525x; 1.6956x over previous
import functools

import jax
import jax.numpy as jnp
from jax import lax
from jax.experimental import pallas as pl
from jax.experimental.pallas import tpu as pltpu

_DN = (((1,), (1,)), ((), ()))


def _bi_rnn_kernel(
        xt_ref, xi_ref, xo_ref,
        wl_t_ref, wl_i_ref, wl_o_ref,
        bl_t_ref, bl_i_ref, bl_o_ref,
        wihf_t_ref, wihf_i_ref, wihf_o_ref,
        wihb_t_ref, wihb_i_ref, wihb_o_ref,
        whhf_t_ref, whhf_i_ref, whhf_o_ref,
        whhb_t_ref, whhb_i_ref, whhb_o_ref,
        bihf_t_ref, bihf_i_ref, bihf_o_ref,
        bhhf_t_ref, bhhf_i_ref, bhhf_o_ref,
        bihb_t_ref, bihb_i_ref, bihb_o_ref,
        bhhb_t_ref, bhhb_i_ref, bhhb_o_ref,
        out_ref,
        xst_scr, xsi_scr, xso_scr, gt_scr, gi_scr, go_scr, whht_scr):
    Bh, T, two_h = out_ref.shape
    H = two_h // 2
    G = 4 * H
    D = xt_ref.shape[2]

    whh_raw = (whhf_t_ref, whhb_t_ref, whhf_i_ref,
               whhb_i_ref, whhf_o_ref, whhb_o_ref)
    for s in range(6):
        whht_scr[s] = jnp.transpose(
            whh_raw[s][...], (1, 0)).astype(jnp.bfloat16)

    def project(x_ref, wl_ref, bl_ref, wihf_ref, wihb_ref,
                bf2, bb2, g_scr, xs_scr):
        for t in range(T):
            xs_scr[t * Bh:(t + 1) * Bh, :] = (
                x_ref[:, t, :].astype(jnp.bfloat16))
        lin = lax.dot_general(
            xs_scr[...], wl_ref[...].astype(jnp.bfloat16), _DN,
            preferred_element_type=jnp.float32)
        lin = (lin + jnp.reshape(bl_ref[...], (1, D))).astype(jnp.bfloat16)
        g_scr[:, 0:G] = lax.dot_general(
            lin, wihf_ref[...].astype(jnp.bfloat16), _DN,
            preferred_element_type=jnp.float32) + bf2
        g_scr[:, G:2 * G] = lax.dot_general(
            lin, wihb_ref[...].astype(jnp.bfloat16), _DN,
            preferred_element_type=jnp.float32) + bb2

    mods = (
        (xt_ref, wl_t_ref, bl_t_ref, wihf_t_ref, wihb_t_ref,
         bihf_t_ref, bhhf_t_ref, bihb_t_ref, bhhb_t_ref, gt_scr, xst_scr),
        (xi_ref, wl_i_ref, bl_i_ref, wihf_i_ref, wihb_i_ref,
         bihf_i_ref, bhhf_i_ref, bihb_i_ref, bhhb_i_ref, gi_scr, xsi_scr),
        (xo_ref, wl_o_ref, bl_o_ref, wihf_o_ref, wihb_o_ref,
         bihf_o_ref, bhhf_o_ref, bihb_o_ref, bhhb_o_ref, go_scr, xso_scr),
    )
    for (x_ref, wl_ref, bl_ref, wihf_ref, wihb_ref,
         bihf_ref, bhhf_ref, bihb_ref, bhhb_ref, g_scr, xs_scr) in mods:
        bf2 = jnp.reshape(bihf_ref[...] + bhhf_ref[...], (1, G))
        bb2 = jnp.reshape(bihb_ref[...] + bhhb_ref[...], (1, G))
        project(x_ref, wl_ref, bl_ref, wihf_ref, wihb_ref, bf2, bb2,
                g_scr, xs_scr)

    g_scrs = (gt_scr, gi_scr, go_scr)

    h = [jnp.zeros((Bh, H), jnp.float32) for _ in range(6)]
    c = [jnp.zeros((Bh, H), jnp.float32) for _ in range(6)]
    inv3 = 1.0 / 3.0

    for t in range(T):
        fwd_sum = jnp.zeros((Bh, H), jnp.float32)
        bwd_sum = jnp.zeros((Bh, H), jnp.float32)
        for m in range(3):
            for d in range(2):
                s = 2 * m + d
                tt = t if d == 0 else T - 1 - t
                gx = g_scrs[m][tt * Bh:(tt + 1) * Bh, d * G:(d + 1) * G]
                gsum = gx + jnp.dot(h[s].astype(jnp.bfloat16), whht_scr[s],
                                    preferred_element_type=jnp.float32)
                i_g = jax.nn.sigmoid(gsum[:, 0:H])
                f_g = jax.nn.sigmoid(gsum[:, H:2 * H])
                g_g = jnp.tanh(gsum[:, 2 * H:3 * H])
                o_g = jax.nn.sigmoid(gsum[:, 3 * H:4 * H])
                c[s] = f_g * c[s] + i_g * g_g
                h[s] = o_g * jnp.tanh(c[s])
                if d == 0:
                    fwd_sum = fwd_sum + h[s]
                else:
                    bwd_sum = bwd_sum + h[s]
        out_ref[:, t:t + 1, 0:H] = (fwd_sum * inv3)[:, None, :]
        out_ref[:, T - 1 - t:T - t, H:2 * H] = (bwd_sum * inv3)[:, None, :]


_NCORES = 2


@functools.partial(jax.jit, static_argnames=())
def _bi_rnn_forward(*args33):
    x_text = args33[0]
    B, T, D = x_text.shape
    H = args33[15].shape[1]
    Bh = B // _NCORES

    def full(shape):
        zeros = (0,) * len(shape)
        return pl.BlockSpec(shape, lambda i, _z=zeros: _z)

    in_specs = [pl.BlockSpec((Bh, T, a.shape[2]), lambda i: (i, 0, 0))
                for a in args33[:3]]
    in_specs += [full(a.shape) for a in args33[3:]]

    return pl.pallas_call(
        _bi_rnn_kernel,
        out_shape=jax.ShapeDtypeStruct((B, T, 2 * H), jnp.float32),
        grid=(_NCORES,),
        in_specs=in_specs,
        out_specs=pl.BlockSpec((Bh, T, 2 * H), lambda i: (i, 0, 0)),
        scratch_shapes=[
            pltpu.VMEM((T * Bh, D), jnp.bfloat16),
            pltpu.VMEM((T * Bh, D), jnp.bfloat16),
            pltpu.VMEM((T * Bh, D), jnp.bfloat16),
            pltpu.VMEM((T * Bh, 8 * H), jnp.float32),
            pltpu.VMEM((T * Bh, 8 * H), jnp.float32),
            pltpu.VMEM((T * Bh, 8 * H), jnp.float32),
            pltpu.VMEM((6, H, 4 * H), jnp.bfloat16),
        ],
        compiler_params=pltpu.CompilerParams(
            dimension_semantics=("parallel",)),
    )(*args33)


def kernel(x_text, x_image, x_other,
           text__w_lin, text__b_lin,
           text__w_ih_f, text__w_hh_f, text__b_ih_f, text__b_hh_f,
           text__w_ih_b, text__w_hh_b, text__b_ih_b, text__b_hh_b,
           image__w_lin, image__b_lin,
           image__w_ih_f, image__w_hh_f, image__b_ih_f, image__b_hh_f,
           image__w_ih_b, image__w_hh_b, image__b_ih_b, image__b_hh_b,
           other__w_lin, other__b_lin,
           other__w_ih_f, other__w_hh_f, other__b_ih_f, other__b_hh_f,
           other__w_ih_b, other__w_hh_b, other__b_ih_b, other__b_hh_b):
    return _bi_rnn_forward(
        x_text, x_image, x_other,
        text__w_lin, image__w_lin, other__w_lin,
        text__b_lin, image__b_lin, other__b_lin,
        text__w_ih_f, image__w_ih_f, other__w_ih_f,
        text__w_ih_b, image__w_ih_b, other__w_ih_b,
        text__w_hh_f, image__w_hh_f, other__w_hh_f,
        text__w_hh_b, image__w_hh_b, other__w_hh_b,
        text__b_ih_f, image__b_ih_f, other__b_ih_f,
        text__b_hh_f, image__b_hh_f, other__b_hh_f,
        text__b_ih_b, image__b_ih_b, other__b_ih_b,
        text__b_hh_b, image__b_hh_b, other__b_hh_b)


# speedup vs baseline: 2.4724x; 1.2046x over previous
import functools

import jax
import jax.numpy as jnp
from jax import lax
from jax.experimental import pallas as pl
from jax.experimental.pallas import tpu as pltpu

_DN = (((1,), (1,)), ((), ()))


def _bi_rnn_kernel(
        xt_ref, xi_ref, xo_ref,
        wl_t_ref, wl_i_ref, wl_o_ref,
        bl_t_ref, bl_i_ref, bl_o_ref,
        wihf_t_ref, wihf_i_ref, wihf_o_ref,
        wihb_t_ref, wihb_i_ref, wihb_o_ref,
        whhf_t_ref, whhf_i_ref, whhf_o_ref,
        whhb_t_ref, whhb_i_ref, whhb_o_ref,
        bihf_t_ref, bihf_i_ref, bihf_o_ref,
        bhhf_t_ref, bhhf_i_ref, bhhf_o_ref,
        bihb_t_ref, bihb_i_ref, bihb_o_ref,
        bhhb_t_ref, bhhb_i_ref, bhhb_o_ref,
        out_ref,
        xst_scr, xsi_scr, xso_scr, gt_scr, gi_scr, go_scr, whht_scr,
        dma_sems):
    Bh, T, two_h = out_ref.shape
    H = two_h // 2
    G = 4 * H
    D = xt_ref.shape[2]
    b0 = pl.program_id(0) * Bh

    xs_scrs = (xst_scr, xsi_scr, xso_scr)
    copies = [[pltpu.make_async_copy(x_hbm.at[pl.ds(b0, Bh), t, :],
                                     xs.at[t], dma_sems.at[mi])
               for t in range(T)]
              for mi, (x_hbm, xs) in enumerate(
                  zip((xt_ref, xi_ref, xo_ref), xs_scrs))]
    for cs in copies:
        for cp in cs:
            cp.start()

    whh_raw = (whhf_t_ref, whhb_t_ref, whhf_i_ref,
               whhb_i_ref, whhf_o_ref, whhb_o_ref)
    for s in range(6):
        whht_scr[s] = jnp.transpose(
            whh_raw[s][...], (1, 0)).astype(jnp.bfloat16)

    def project(mi, wl_ref, bl_ref, wihf_ref, wihb_ref, bf2, bb2, g_scr):
        for cp in copies[mi]:
            cp.wait()
        lin = lax.dot_general(
            xs_scrs[mi][...].reshape(T * Bh, D), wl_ref[...], _DN,
            preferred_element_type=jnp.float32)
        lin = (lin + jnp.reshape(bl_ref[...], (1, D))).astype(jnp.bfloat16)
        g_scr[:, 0:G] = lax.dot_general(
            lin, wihf_ref[...].astype(jnp.bfloat16), _DN,
            preferred_element_type=jnp.float32) + bf2
        g_scr[:, G:2 * G] = lax.dot_general(
            lin, wihb_ref[...].astype(jnp.bfloat16), _DN,
            preferred_element_type=jnp.float32) + bb2

    mods = (
        (wl_t_ref, bl_t_ref, wihf_t_ref, wihb_t_ref,
         bihf_t_ref, bhhf_t_ref, bihb_t_ref, bhhb_t_ref, gt_scr),
        (wl_i_ref, bl_i_ref, wihf_i_ref, wihb_i_ref,
         bihf_i_ref, bhhf_i_ref, bihb_i_ref, bhhb_i_ref, gi_scr),
        (wl_o_ref, bl_o_ref, wihf_o_ref, wihb_o_ref,
         bihf_o_ref, bhhf_o_ref, bihb_o_ref, bhhb_o_ref, go_scr),
    )
    for mi, (wl_ref, bl_ref, wihf_ref, wihb_ref,
             bihf_ref, bhhf_ref, bihb_ref, bhhb_ref, g_scr) in enumerate(mods):
        bf2 = jnp.reshape(bihf_ref[...] + bhhf_ref[...], (1, G))
        bb2 = jnp.reshape(bihb_ref[...] + bhhb_ref[...], (1, G))
        project(mi, wl_ref, bl_ref, wihf_ref, wihb_ref, bf2, bb2, g_scr)

    g_scrs = (gt_scr, gi_scr, go_scr)

    h = [jnp.zeros((Bh, H), jnp.float32) for _ in range(6)]
    c = [jnp.zeros((Bh, H), jnp.float32) for _ in range(6)]
    inv3 = 1.0 / 3.0

    for t in range(T):
        fwd_sum = jnp.zeros((Bh, H), jnp.float32)
        bwd_sum = jnp.zeros((Bh, H), jnp.float32)
        for m in range(3):
            for d in range(2):
                s = 2 * m + d
                tt = t if d == 0 else T - 1 - t
                gx = g_scrs[m][tt * Bh:(tt + 1) * Bh, d * G:(d + 1) * G]
                gsum = gx + jnp.dot(h[s].astype(jnp.bfloat16), whht_scr[s],
                                    preferred_element_type=jnp.float32)
                i_g = jax.nn.sigmoid(gsum[:, 0:H])
                f_g = jax.nn.sigmoid(gsum[:, H:2 * H])
                g_g = jnp.tanh(gsum[:, 2 * H:3 * H])
                o_g = jax.nn.sigmoid(gsum[:, 3 * H:4 * H])
                c[s] = f_g * c[s] + i_g * g_g
                h[s] = o_g * jnp.tanh(c[s])
                if d == 0:
                    fwd_sum = fwd_sum + h[s]
                else:
                    bwd_sum = bwd_sum + h[s]
        out_ref[:, t:t + 1, 0:H] = (fwd_sum * inv3)[:, None, :]
        out_ref[:, T - 1 - t:T - t, H:2 * H] = (bwd_sum * inv3)[:, None, :]


_NCORES = 2


@functools.partial(jax.jit, static_argnames=())
def _bi_rnn_forward(*args33):
    x_text = args33[0]
    B, T, D = x_text.shape
    H = args33[15].shape[1]
    Bh = B // _NCORES

    def full(shape):
        zeros = (0,) * len(shape)
        return pl.BlockSpec(shape, lambda i, _z=zeros: _z)

    in_specs = [pl.BlockSpec(memory_space=pl.ANY) for _ in args33[:3]]
    in_specs += [full(a.shape) for a in args33[3:]]

    return pl.pallas_call(
        _bi_rnn_kernel,
        out_shape=jax.ShapeDtypeStruct((B, T, 2 * H), jnp.float32),
        grid=(_NCORES,),
        in_specs=in_specs,
        out_specs=pl.BlockSpec((Bh, T, 2 * H), lambda i: (i, 0, 0)),
        scratch_shapes=[
            pltpu.VMEM((T, Bh, D), jnp.float32),
            pltpu.VMEM((T, Bh, D), jnp.float32),
            pltpu.VMEM((T, Bh, D), jnp.float32),
            pltpu.VMEM((T * Bh, 8 * H), jnp.float32),
            pltpu.VMEM((T * Bh, 8 * H), jnp.float32),
            pltpu.VMEM((T * Bh, 8 * H), jnp.float32),
            pltpu.VMEM((6, H, 4 * H), jnp.bfloat16),
            pltpu.SemaphoreType.DMA((3,)),
        ],
        compiler_params=pltpu.CompilerParams(
            dimension_semantics=("parallel",)),
    )(*args33)


def kernel(x_text, x_image, x_other,
           text__w_lin, text__b_lin,
           text__w_ih_f, text__w_hh_f, text__b_ih_f, text__b_hh_f,
           text__w_ih_b, text__w_hh_b, text__b_ih_b, text__b_hh_b,
           image__w_lin, image__b_lin,
           image__w_ih_f, image__w_hh_f, image__b_ih_f, image__b_hh_f,
           image__w_ih_b, image__w_hh_b, image__b_ih_b, image__b_hh_b,
           other__w_lin, other__b_lin,
           other__w_ih_f, other__w_hh_f, other__b_ih_f, other__b_hh_f,
           other__w_ih_b, other__w_hh_b, other__b_ih_b, other__b_hh_b):
    return _bi_rnn_forward(
        x_text, x_image, x_other,
        text__w_lin, image__w_lin, other__w_lin,
        text__b_lin, image__b_lin, other__b_lin,
        text__w_ih_f, image__w_ih_f, other__w_ih_f,
        text__w_ih_b, image__w_ih_b, other__w_ih_b,
        text__w_hh_f, image__w_hh_f, other__w_hh_f,
        text__w_hh_b, image__w_hh_b, other__w_hh_b,
        text__b_ih_f, image__b_ih_f, other__b_ih_f,
        text__b_hh_f, image__b_hh_f, other__b_hh_f,
        text__b_ih_b, image__b_ih_b, other__b_ih_b,
        text__b_hh_b, image__b_hh_b, other__b_hh_b)
